# baseline (speedup 1.0000x reference)
import functools

import jax
import jax.numpy as jnp
from jax import lax
from jax.experimental import pallas as pl
from jax.experimental.pallas import tpu as pltpu
from jax.experimental.pallas import tpu_sc as plsc

D = 2048
F = 1408
E = 16
SF = 5632
T = 2048

B = 256
NB = 32
NSLOTS = NB * B

NEG_INF = -1e30


def _router_body(x_ref, gw_ref, sgw_ref,
                 p0_ref, p1_ref, w0_ref, w1_ref, sig_ref, bexp_ref, valid_ref,
                 nv_ref):
    x = x_ref[...]
    logits = jnp.dot(x, gw_ref[...], preferred_element_type=jnp.float32)
    m = jnp.max(logits, axis=-1, keepdims=True)
    p = jnp.exp(logits - m)
    p = p / jnp.sum(p, axis=-1, keepdims=True)
    lane = lax.broadcasted_iota(jnp.int32, (T, E), 1)
    p1v = jnp.max(p, axis=-1, keepdims=True)
    e0 = jnp.min(jnp.where(p == p1v, lane, E), axis=-1, keepdims=True)
    pm = jnp.where(lane == e0, NEG_INF, p)
    p2v = jnp.max(pm, axis=-1, keepdims=True)
    e1 = jnp.min(jnp.where(pm == p2v, lane, E), axis=-1, keepdims=True)
    s = p1v + p2v
    w0_ref[...] = p1v / s
    w1_ref[...] = p2v / s

    mask = (jnp.where(lane == e0, 1.0, 0.0) + jnp.where(lane == e1, 1.0, 0.0))
    r_t = lax.broadcasted_iota(jnp.int32, (T, T), 0)
    c_t = lax.broadcasted_iota(jnp.int32, (T, T), 1)
    lt = jnp.where(c_t < r_t, 1.0, 0.0)
    ranks = jnp.dot(lt, mask, preferred_element_type=jnp.float32)

    counts = jnp.sum(mask, axis=0, keepdims=True)
    nb = jnp.floor((counts + (B - 1.0)) * (1.0 / B))
    laneE = lax.broadcasted_iota(jnp.int32, (1, E), 1)
    r_e = lax.broadcasted_iota(jnp.int32, (E, E), 0)
    c_e = lax.broadcasted_iota(jnp.int32, (E, E), 1)
    lt16 = jnp.where(r_e < c_e, 1.0, 0.0)
    bstart = jnp.dot(nb, lt16, preferred_element_type=jnp.float32)
    bend = bstart + nb
    off = bstart * float(B)

    sel0 = jnp.where(lane == e0, 1.0, 0.0)
    sel1 = jnp.where(lane == e1, 1.0, 0.0)
    p0 = jnp.sum(sel0 * (off + ranks), axis=-1, keepdims=True)
    p1 = jnp.sum(sel1 * (off + ranks), axis=-1, keepdims=True)
    p0_ref[...] = p0.astype(jnp.int32)
    p1_ref[...] = p1.astype(jnp.int32)

    bi = lax.broadcasted_iota(jnp.int32, (NB, E), 0)
    bend_i = bend.astype(jnp.int32)
    braw = jnp.sum(jnp.where(bi >= bend_i, 1, 0), axis=-1, keepdims=True)
    lastexp = jnp.max(jnp.where(nb > 0, laneE, 0))
    bexp_ref[...] = jnp.minimum(braw, lastexp)
    nbtot = jnp.sum(nb).astype(jnp.int32)
    bvec = lax.broadcasted_iota(jnp.int32, (NB, 1), 0)
    valid_ref[...] = jnp.where(bvec < nbtot, 1, 0).astype(jnp.int32)
    nv_ref[...] = jnp.full((1, 16), B, jnp.int32) * nbtot

    sig_ref[...] = jax.nn.sigmoid(
        jnp.dot(x, sgw_ref[...], preferred_element_type=jnp.float32))


def _router(x, gate_w, shared_gate_w):
    outs = pl.pallas_call(
        _router_body,
        out_shape=(
            jax.ShapeDtypeStruct((T, 1), jnp.int32),
            jax.ShapeDtypeStruct((T, 1), jnp.int32),
            jax.ShapeDtypeStruct((T, 1), jnp.float32),
            jax.ShapeDtypeStruct((T, 1), jnp.float32),
            jax.ShapeDtypeStruct((T, 1), jnp.float32),
            jax.ShapeDtypeStruct((NB, 1), jnp.int32),
            jax.ShapeDtypeStruct((NB, 1), jnp.int32),
            jax.ShapeDtypeStruct((1, 16), jnp.int32),
        ),
    )(x, gate_w, shared_gate_w)
    return outs


def _sc_mesh():
    return plsc.VectorSubcoreMesh(core_axis_name="c", subcore_axis_name="s")


NW = 32
SPT = NSLOTS // NW
GCH = 16
NCH = SPT // GCH


def _dispatch_body(p0_hbm, p1_hbm, x_hbm, nv_hbm, xg_hbm,
                   p0_v, p1_v, table_v, nv_v, rows_v, gsem, osem):
    wid = lax.axis_index("s") * 2 + lax.axis_index("c")
    pltpu.sync_copy(p0_hbm, p0_v)
    pltpu.sync_copy(p1_hbm, p1_v)
    pltpu.sync_copy(nv_hbm, nv_v)
    nvalid = jnp.max(nv_v[...])

    def zero_step(i, _):
        table_v[pl.ds(i * 16, 16)] = jnp.zeros((16,), jnp.int32)
        return 0

    lax.fori_loop(0, NSLOTS // 16, zero_step, 0)

    def scat0(j, _):
        idx = p0_v[pl.ds(j * 16, 16)]
        toks = lax.iota(jnp.int32, 16) + j * 16
        plsc.store_scatter(table_v, [idx], toks)
        return 0

    def scat1(j, _):
        idx = p1_v[pl.ds(j * 16, 16)]
        toks = lax.iota(jnp.int32, 16) + j * 16
        plsc.store_scatter(table_v, [idx], toks)
        return 0

    lax.fori_loop(0, T // 16, scat0, 0)
    lax.fori_loop(0, T // 16, scat1, 0)

    base0 = wid * SPT

    def live(c):
        return (base0 + c * GCH) < nvalid

    def start_gather(c):
        b = c % 2
        idx = table_v[pl.ds(base0 + c * GCH, GCH)]
        pltpu.async_copy(x_hbm.at[idx], rows_v.at[b], gsem.at[b])

    @pl.when(live(0))
    def _():
        start_gather(0)

    for c in range(NCH):
        b = c % 2
        if c >= 1:

            @pl.when(live(c - 1))
            def _(c=c):
                pltpu.make_async_copy(rows_v.at[1 - b],
                                      xg_hbm.at[pl.ds(base0 + (c - 1) * GCH,
                                                      GCH)],
                                      osem.at[1 - b]).wait()
        if c + 1 < NCH:

            @pl.when(live(c + 1))
            def _(c=c):
                start_gather(c + 1)

        @pl.when(live(c))
        def _(c=c):
            pltpu.make_async_copy(x_hbm.at[table_v[pl.ds(0, GCH)]],
                                  rows_v.at[b], gsem.at[b]).wait()
            pltpu.async_copy(rows_v.at[b],
                             xg_hbm.at[pl.ds(base0 + c * GCH, GCH)],
                             osem.at[b])

    @pl.when(live(NCH - 1))
    def _():
        pltpu.make_async_copy(rows_v.at[(NCH - 1) % 2],
                              xg_hbm.at[pl.ds(base0 + (NCH - 1) * GCH, GCH)],
                              osem.at[(NCH - 1) % 2]).wait()


def _dispatch_sc(p0, p1, x, nv):
    k = functools.partial(
        pl.kernel,
        out_type=jax.ShapeDtypeStruct((NSLOTS, D), jnp.float32),
        mesh=_sc_mesh(),
        scratch_types=[
            pltpu.VMEM((T,), jnp.int32),
            pltpu.VMEM((T,), jnp.int32),
            pltpu.VMEM((NSLOTS,), jnp.int32),
            pltpu.VMEM((16,), jnp.int32),
            pltpu.VMEM((2, GCH, D), jnp.float32),
            pltpu.SemaphoreType.DMA((2,)),
            pltpu.SemaphoreType.DMA((2,)),
        ],
        compiler_params=pltpu.CompilerParams(needs_layout_passes=False),
    )(_dispatch_body)
    return k(p0, p1, x, nv)


def _moe_h_body(bexp_ref, valid_ref, xg_ref, w1_ref, w3_ref, h_ref,
                h1_s, h3_s):
    i = pl.program_id(0)
    k = pl.program_id(1)

    @pl.when(valid_ref[i] == 1)
    def _():
        xb = xg_ref[...].astype(jnp.bfloat16)
        a1 = jnp.dot(xb, w1_ref[0].astype(jnp.bfloat16),
                     preferred_element_type=jnp.float32)
        a3 = jnp.dot(xb, w3_ref[0].astype(jnp.bfloat16),
                     preferred_element_type=jnp.float32)

        @pl.when(k == 0)
        def _():
            h1_s[...] = a1
            h3_s[...] = a3

        @pl.when(k != 0)
        def _():
            h1_s[...] += a1
            h3_s[...] += a3

        @pl.when(k == 1)
        def _():
            h_ref[...] = jax.nn.silu(h1_s[...]) * h3_s[...]


def _moe_y_body(bexp_ref, valid_ref, h_ref, w2_ref, y_ref):
    i = pl.program_id(0)

    @pl.when(valid_ref[i] == 1)
    def _():
        y_ref[...] = jnp.dot(h_ref[...].astype(jnp.bfloat16),
                             w2_ref[0].astype(jnp.bfloat16),
                             preferred_element_type=jnp.float32)


def _moe_gemm(bexp, valid, xg, W1, W3, W2):
    DK = D // 2
    H = pl.pallas_call(
        _moe_h_body,
        grid_spec=pltpu.PrefetchScalarGridSpec(
            num_scalar_prefetch=2,
            grid=(NB, 2),
            in_specs=[
                pl.BlockSpec((B, DK), lambda i, k, be, va: (i, k)),
                pl.BlockSpec((1, DK, F), lambda i, k, be, va: (be[i], k, 0)),
                pl.BlockSpec((1, DK, F), lambda i, k, be, va: (be[i], k, 0)),
            ],
            out_specs=pl.BlockSpec((B, F), lambda i, k, be, va: (i, 0)),
            scratch_shapes=[
                pltpu.VMEM((B, F), jnp.float32),
                pltpu.VMEM((B, F), jnp.float32),
            ],
        ),
        out_shape=jax.ShapeDtypeStruct((NSLOTS, F), jnp.float32),
        compiler_params=pltpu.CompilerParams(
            dimension_semantics=("arbitrary", "arbitrary"),
        ),
    )(bexp, valid, xg, W1, W3)
    y = pl.pallas_call(
        _moe_y_body,
        grid_spec=pltpu.PrefetchScalarGridSpec(
            num_scalar_prefetch=2,
            grid=(NB,),
            in_specs=[
                pl.BlockSpec((B, F), lambda i, be, va: (i, 0)),
                pl.BlockSpec((1, F, D), lambda i, be, va: (be[i], 0, 0)),
            ],
            out_specs=pl.BlockSpec((B, D), lambda i, be, va: (i, 0)),
        ),
        out_shape=jax.ShapeDtypeStruct((NSLOTS, D), jnp.float32),
        compiler_params=pltpu.CompilerParams(
            dimension_semantics=("arbitrary",),
        ),
    )(bexp, valid, H, W2)
    return y


TPT = T // NW


def _combine_body(p0_hbm, p1_hbm, y_hbm, yg0_hbm, yg1_hbm,
                  p0s, p1s, rows_v, gsem, osem):
    wid = lax.axis_index("s") * 2 + lax.axis_index("c")
    tbase = wid * TPT
    pltpu.sync_copy(p0_hbm.at[pl.ds(tbase, TPT)], p0s)
    pltpu.sync_copy(p1_hbm.at[pl.ds(tbase, TPT)], p1s)

    nq = 2 * (TPT // GCH)

    def chunk(q):
        ps, yg = (p0s, yg0_hbm) if q < nq // 2 else (p1s, yg1_hbm)
        c = q % (nq // 2)
        return ps, yg, c * GCH

    def start_gather(q):
        ps, yg, o = chunk(q)
        b = q % 2
        idx = ps[pl.ds(o, GCH)]
        pltpu.async_copy(y_hbm.at[idx], rows_v.at[b], gsem.at[b])

    start_gather(0)
    for q in range(nq):
        b = q % 2
        if q >= 1:
            psp, ygp, op = chunk(q - 1)
            pltpu.make_async_copy(rows_v.at[1 - b],
                                  ygp.at[pl.ds(tbase + op, GCH)],
                                  osem.at[1 - b]).wait()
        if q + 1 < nq:
            start_gather(q + 1)
        ps, yg, o = chunk(q)
        pltpu.make_async_copy(y_hbm.at[ps[pl.ds(0, GCH)]],
                              rows_v.at[b], gsem.at[b]).wait()
        pltpu.async_copy(rows_v.at[b], yg.at[pl.ds(tbase + o, GCH)],
                         osem.at[b])
    psl, ygl, ol = chunk(nq - 1)
    pltpu.make_async_copy(rows_v.at[(nq - 1) % 2],
                          ygl.at[pl.ds(tbase + ol, GCH)],
                          osem.at[(nq - 1) % 2]).wait()


def _combine_sc(p0, p1, y):
    k = functools.partial(
        pl.kernel,
        out_type=(
            jax.ShapeDtypeStruct((T, D), jnp.float32),
            jax.ShapeDtypeStruct((T, D), jnp.float32),
        ),
        mesh=_sc_mesh(),
        scratch_types=[
            pltpu.VMEM((TPT,), jnp.int32),
            pltpu.VMEM((TPT,), jnp.int32),
            pltpu.VMEM((2, GCH, D), jnp.float32),
            pltpu.SemaphoreType.DMA((2,)),
            pltpu.SemaphoreType.DMA((2,)),
        ],
        compiler_params=pltpu.CompilerParams(needs_layout_passes=False),
    )(_combine_body)
    return k(p0, p1, y)


def _shared_h_body(x_ref, ws1_ref, ws3_ref, h_ref):
    x = x_ref[...].astype(jnp.bfloat16)
    a = jnp.dot(x, ws1_ref[...].astype(jnp.bfloat16),
                preferred_element_type=jnp.float32)
    b = jnp.dot(x, ws3_ref[...].astype(jnp.bfloat16),
                preferred_element_type=jnp.float32)
    h_ref[...] = jax.nn.silu(a) * b


def _shared_out_body(h_ref, ws2_ref, sig_ref, w0_ref, w1_ref,
                     yg0_ref, yg1_ref, out_ref, *, nfs):
    j = pl.program_id(1)
    part = jnp.dot(h_ref[...].astype(jnp.bfloat16),
                   ws2_ref[...].astype(jnp.bfloat16),
                   preferred_element_type=jnp.float32)

    @pl.when(j == 0)
    def _():
        out_ref[...] = part

    @pl.when(j != 0)
    def _():
        out_ref[...] += part

    @pl.when(j == nfs - 1)
    def _():
        out_ref[...] = (sig_ref[...] * out_ref[...]
                        + w0_ref[...] * yg0_ref[...]
                        + w1_ref[...] * yg1_ref[...])


def _shared_h(x, Ws1, Ws3):
    FS = 512
    nfs = SF // FS
    Hs = pl.pallas_call(
        _shared_h_body,
        grid=(nfs,),
        in_specs=[
            pl.BlockSpec((T, D), lambda j: (0, 0)),
            pl.BlockSpec((D, FS), lambda j: (0, j)),
            pl.BlockSpec((D, FS), lambda j: (0, j)),
        ],
        out_specs=pl.BlockSpec((T, FS), lambda j: (0, j)),
        out_shape=jax.ShapeDtypeStruct((T, SF), jnp.float32),
        compiler_params=pltpu.CompilerParams(
            dimension_semantics=("arbitrary",),
            vmem_limit_bytes=112 * 1024 * 1024,
        ),
    )(x, Ws1, Ws3)
    return Hs


def _shared_out(Hs, Ws2, sig, w0, w1, yg0, yg1):
    FS = 512
    nfs = SF // FS
    BT2 = 512
    out = pl.pallas_call(
        functools.partial(_shared_out_body, nfs=nfs),
        grid=(T // BT2, nfs),
        in_specs=[
            pl.BlockSpec((BT2, FS), lambda t, j: (t, j)),
            pl.BlockSpec((FS, D), lambda t, j: (j, 0)),
            pl.BlockSpec((BT2, 1), lambda t, j: (t, 0)),
            pl.BlockSpec((BT2, 1), lambda t, j: (t, 0)),
            pl.BlockSpec((BT2, 1), lambda t, j: (t, 0)),
            pl.BlockSpec((BT2, D), lambda t, j: (t, 0)),
            pl.BlockSpec((BT2, D), lambda t, j: (t, 0)),
        ],
        out_specs=pl.BlockSpec((BT2, D), lambda t, j: (t, 0)),
        out_shape=jax.ShapeDtypeStruct((T, D), jnp.float32),
        compiler_params=pltpu.CompilerParams(
            dimension_semantics=("arbitrary", "arbitrary"),
        ),
    )(Hs, Ws2, sig, w0, w1, yg0, yg1)
    return out


def kernel(x, gate_w, shared_gate_w, Ws1, Ws3, Ws2, W1, W3, W2):
    p0, p1, w0, w1, sig, bexp, valid, nv = _router(x, gate_w, shared_gate_w)
    p0f = p0.reshape(T)
    p1f = p1.reshape(T)
    xg = _dispatch_sc(p0f, p1f, x, nv.reshape(16))
    Hs = _shared_h(x, Ws1, Ws3)
    y = _moe_gemm(bexp.reshape(NB), valid.reshape(NB), xg, W1, W3, W2)
    yg0, yg1 = _combine_sc(p0f, p1f, y)
    out = _shared_out(Hs, Ws2, sig, w0, w1, yg0, yg1)
    return out


# speedup vs baseline: 1.1997x; 1.1997x over previous
import functools

import jax
import jax.numpy as jnp
from jax import lax
from jax.experimental import pallas as pl
from jax.experimental.pallas import tpu as pltpu
from jax.experimental.pallas import tpu_sc as plsc

D = 2048
F = 1408
E = 16
SF = 5632
T = 2048

B = 256
NB = 32
NSLOTS = NB * B

NEG_INF = -1e30


def _router_body(x_ref, gw_ref, sgw_ref,
                 p0_ref, p1_ref, w0_ref, w1_ref, sig_ref, bexp_ref, valid_ref,
                 nv_ref):
    x = x_ref[...]
    logits = jnp.dot(x, gw_ref[...], preferred_element_type=jnp.float32)
    m = jnp.max(logits, axis=-1, keepdims=True)
    p = jnp.exp(logits - m)
    p = p / jnp.sum(p, axis=-1, keepdims=True)
    lane = lax.broadcasted_iota(jnp.int32, (T, E), 1)
    p1v = jnp.max(p, axis=-1, keepdims=True)
    e0 = jnp.min(jnp.where(p == p1v, lane, E), axis=-1, keepdims=True)
    pm = jnp.where(lane == e0, NEG_INF, p)
    p2v = jnp.max(pm, axis=-1, keepdims=True)
    e1 = jnp.min(jnp.where(pm == p2v, lane, E), axis=-1, keepdims=True)
    s = p1v + p2v
    w0_ref[...] = p1v / s
    w1_ref[...] = p2v / s

    mask = (jnp.where(lane == e0, 1.0, 0.0) + jnp.where(lane == e1, 1.0, 0.0))
    r_t = lax.broadcasted_iota(jnp.int32, (T, T), 0)
    c_t = lax.broadcasted_iota(jnp.int32, (T, T), 1)
    lt = jnp.where(c_t < r_t, 1.0, 0.0)
    ranks = jnp.dot(lt, mask, preferred_element_type=jnp.float32)

    counts = jnp.sum(mask, axis=0, keepdims=True)
    nb = jnp.floor((counts + (B - 1.0)) * (1.0 / B))
    laneE = lax.broadcasted_iota(jnp.int32, (1, E), 1)
    r_e = lax.broadcasted_iota(jnp.int32, (E, E), 0)
    c_e = lax.broadcasted_iota(jnp.int32, (E, E), 1)
    lt16 = jnp.where(r_e < c_e, 1.0, 0.0)
    bstart = jnp.dot(nb, lt16, preferred_element_type=jnp.float32)
    bend = bstart + nb
    off = bstart * float(B)

    sel0 = jnp.where(lane == e0, 1.0, 0.0)
    sel1 = jnp.where(lane == e1, 1.0, 0.0)
    p0 = jnp.sum(sel0 * (off + ranks), axis=-1, keepdims=True)
    p1 = jnp.sum(sel1 * (off + ranks), axis=-1, keepdims=True)
    p0_ref[...] = p0.astype(jnp.int32)
    p1_ref[...] = p1.astype(jnp.int32)

    bi = lax.broadcasted_iota(jnp.int32, (NB, E), 0)
    bend_i = bend.astype(jnp.int32)
    braw = jnp.sum(jnp.where(bi >= bend_i, 1, 0), axis=-1, keepdims=True)
    lastexp = jnp.max(jnp.where(nb > 0, laneE, 0))
    bexp_ref[...] = jnp.minimum(braw, lastexp)
    nbtot = jnp.sum(nb).astype(jnp.int32)
    bvec = lax.broadcasted_iota(jnp.int32, (NB, 1), 0)
    valid_ref[...] = jnp.where(bvec < nbtot, 1, 0).astype(jnp.int32)
    nv_ref[...] = jnp.full((1, 16), B, jnp.int32) * nbtot

    sig_ref[...] = jax.nn.sigmoid(
        jnp.dot(x, sgw_ref[...], preferred_element_type=jnp.float32))


def _router(x, gate_w, shared_gate_w):
    outs = pl.pallas_call(
        _router_body,
        out_shape=(
            jax.ShapeDtypeStruct((T, 1), jnp.int32),
            jax.ShapeDtypeStruct((T, 1), jnp.int32),
            jax.ShapeDtypeStruct((T, 1), jnp.float32),
            jax.ShapeDtypeStruct((T, 1), jnp.float32),
            jax.ShapeDtypeStruct((T, 1), jnp.float32),
            jax.ShapeDtypeStruct((NB, 1), jnp.int32),
            jax.ShapeDtypeStruct((NB, 1), jnp.int32),
            jax.ShapeDtypeStruct((1, 16), jnp.int32),
        ),
    )(x, gate_w, shared_gate_w)
    return outs


def _sc_mesh():
    return plsc.VectorSubcoreMesh(core_axis_name="c", subcore_axis_name="s")


NW = 32
SPT = NSLOTS // NW
GCH = 16
NCH = SPT // GCH


def _dispatch_body(p0_hbm, p1_hbm, x_hbm, nv_hbm, xg_hbm,
                   p0_v, p1_v, table_v, nv_v, rows_v, gsem, osem):
    wid = lax.axis_index("s") * 2 + lax.axis_index("c")
    pltpu.sync_copy(p0_hbm, p0_v)
    pltpu.sync_copy(p1_hbm, p1_v)
    pltpu.sync_copy(nv_hbm, nv_v)
    nvalid = jnp.max(nv_v[...])

    def zero_step(i, _):
        table_v[pl.ds(i * 16, 16)] = jnp.zeros((16,), jnp.int32)
        return 0

    lax.fori_loop(0, NSLOTS // 16, zero_step, 0)

    def scat0(j, _):
        idx = p0_v[pl.ds(j * 16, 16)]
        toks = lax.iota(jnp.int32, 16) + j * 16
        plsc.store_scatter(table_v, [idx], toks)
        return 0

    def scat1(j, _):
        idx = p1_v[pl.ds(j * 16, 16)]
        toks = lax.iota(jnp.int32, 16) + j * 16
        plsc.store_scatter(table_v, [idx], toks)
        return 0

    lax.fori_loop(0, T // 16, scat0, 0)
    lax.fori_loop(0, T // 16, scat1, 0)

    base0 = wid * SPT

    def live(c):
        return (base0 + c * GCH) < nvalid

    def start_gather(c):
        b = c % 2
        idx = table_v[pl.ds(base0 + c * GCH, GCH)]
        pltpu.async_copy(x_hbm.at[idx], rows_v.at[b], gsem.at[b])

    @pl.when(live(0))
    def _():
        start_gather(0)

    for c in range(NCH):
        b = c % 2
        if c >= 1:

            @pl.when(live(c - 1))
            def _(c=c):
                pltpu.make_async_copy(rows_v.at[1 - b],
                                      xg_hbm.at[pl.ds(base0 + (c - 1) * GCH,
                                                      GCH)],
                                      osem.at[1 - b]).wait()
        if c + 1 < NCH:

            @pl.when(live(c + 1))
            def _(c=c):
                start_gather(c + 1)

        @pl.when(live(c))
        def _(c=c):
            pltpu.make_async_copy(x_hbm.at[table_v[pl.ds(0, GCH)]],
                                  rows_v.at[b], gsem.at[b]).wait()
            pltpu.async_copy(rows_v.at[b],
                             xg_hbm.at[pl.ds(base0 + c * GCH, GCH)],
                             osem.at[b])

    @pl.when(live(NCH - 1))
    def _():
        pltpu.make_async_copy(rows_v.at[(NCH - 1) % 2],
                              xg_hbm.at[pl.ds(base0 + (NCH - 1) * GCH, GCH)],
                              osem.at[(NCH - 1) % 2]).wait()


def _dispatch_sc(p0, p1, x, nv):
    k = functools.partial(
        pl.kernel,
        out_type=jax.ShapeDtypeStruct((NSLOTS, D), jnp.float32),
        mesh=_sc_mesh(),
        scratch_types=[
            pltpu.VMEM((T,), jnp.int32),
            pltpu.VMEM((T,), jnp.int32),
            pltpu.VMEM((NSLOTS,), jnp.int32),
            pltpu.VMEM((16,), jnp.int32),
            pltpu.VMEM((2, GCH, D), jnp.float32),
            pltpu.SemaphoreType.DMA((2,)),
            pltpu.SemaphoreType.DMA((2,)),
        ],
        compiler_params=pltpu.CompilerParams(needs_layout_passes=False),
    )(_dispatch_body)
    return k(p0, p1, x, nv)


def _moe_h_body(bexp_ref, valid_ref, hdep_ref, xg_ref, w1_ref, w3_ref, h_ref):
    i = pl.program_id(0)

    @pl.when(valid_ref[i] == 1)
    def _():
        xb = xg_ref[...].astype(jnp.bfloat16)
        a1 = jnp.dot(xb, w1_ref[0].astype(jnp.bfloat16),
                     preferred_element_type=jnp.float32)
        a3 = jnp.dot(xb, w3_ref[0].astype(jnp.bfloat16),
                     preferred_element_type=jnp.float32)
        h_ref[...] = jax.nn.silu(a1) * a3


def _moe_y_body(bexp_ref, valid_ref, h_ref, w2_ref, y_ref):
    i = pl.program_id(0)

    @pl.when(valid_ref[i] == 1)
    def _():
        y_ref[...] = jnp.dot(h_ref[...].astype(jnp.bfloat16),
                             w2_ref[0].astype(jnp.bfloat16),
                             preferred_element_type=jnp.float32)


def _moe_gemm(bexp, valid, hdep, xg, W1, W3, W2):
    H = pl.pallas_call(
        _moe_h_body,
        grid_spec=pltpu.PrefetchScalarGridSpec(
            num_scalar_prefetch=2,
            grid=(NB,),
            in_specs=[
                pl.BlockSpec((8, 128), lambda i, be, va: (0, 0)),
                pl.BlockSpec((B, D), lambda i, be, va: (i, 0)),
                pl.BlockSpec((1, D, F), lambda i, be, va: (be[i], 0, 0)),
                pl.BlockSpec((1, D, F), lambda i, be, va: (be[i], 0, 0)),
            ],
            out_specs=pl.BlockSpec((B, F), lambda i, be, va: (i, 0)),
        ),
        out_shape=jax.ShapeDtypeStruct((NSLOTS, F), jnp.float32),
        compiler_params=pltpu.CompilerParams(
            dimension_semantics=("arbitrary",),
            vmem_limit_bytes=63 * 1024 * 1024,
        ),
    )(bexp, valid, hdep, xg, W1, W3)
    y = pl.pallas_call(
        _moe_y_body,
        grid_spec=pltpu.PrefetchScalarGridSpec(
            num_scalar_prefetch=2,
            grid=(NB,),
            in_specs=[
                pl.BlockSpec((B, F), lambda i, be, va: (i, 0)),
                pl.BlockSpec((1, F, D), lambda i, be, va: (be[i], 0, 0)),
            ],
            out_specs=pl.BlockSpec((B, D), lambda i, be, va: (i, 0)),
        ),
        out_shape=jax.ShapeDtypeStruct((NSLOTS, D), jnp.float32),
        compiler_params=pltpu.CompilerParams(
            dimension_semantics=("arbitrary",),
            vmem_limit_bytes=63 * 1024 * 1024,
        ),
    )(bexp, valid, H, W2)
    return y


TPT = T // NW


def _combine_body(p0_hbm, p1_hbm, y_hbm, yg0_hbm, yg1_hbm,
                  p0s, p1s, rows_v, gsem, osem):
    wid = lax.axis_index("s") * 2 + lax.axis_index("c")
    tbase = wid * TPT
    pltpu.sync_copy(p0_hbm.at[pl.ds(tbase, TPT)], p0s)
    pltpu.sync_copy(p1_hbm.at[pl.ds(tbase, TPT)], p1s)

    nq = 2 * (TPT // GCH)

    def chunk(q):
        ps, yg = (p0s, yg0_hbm) if q < nq // 2 else (p1s, yg1_hbm)
        c = q % (nq // 2)
        return ps, yg, c * GCH

    def start_gather(q):
        ps, yg, o = chunk(q)
        b = q % 2
        idx = ps[pl.ds(o, GCH)]
        pltpu.async_copy(y_hbm.at[idx], rows_v.at[b], gsem.at[b])

    start_gather(0)
    for q in range(nq):
        b = q % 2
        if q >= 1:
            psp, ygp, op = chunk(q - 1)
            pltpu.make_async_copy(rows_v.at[1 - b],
                                  ygp.at[pl.ds(tbase + op, GCH)],
                                  osem.at[1 - b]).wait()
        if q + 1 < nq:
            start_gather(q + 1)
        ps, yg, o = chunk(q)
        pltpu.make_async_copy(y_hbm.at[ps[pl.ds(0, GCH)]],
                              rows_v.at[b], gsem.at[b]).wait()
        pltpu.async_copy(rows_v.at[b], yg.at[pl.ds(tbase + o, GCH)],
                         osem.at[b])
    psl, ygl, ol = chunk(nq - 1)
    pltpu.make_async_copy(rows_v.at[(nq - 1) % 2],
                          ygl.at[pl.ds(tbase + ol, GCH)],
                          osem.at[(nq - 1) % 2]).wait()


def _combine_sc(p0, p1, y):
    k = functools.partial(
        pl.kernel,
        out_type=(
            jax.ShapeDtypeStruct((T, D), jnp.float32),
            jax.ShapeDtypeStruct((T, D), jnp.float32),
        ),
        mesh=_sc_mesh(),
        scratch_types=[
            pltpu.VMEM((TPT,), jnp.int32),
            pltpu.VMEM((TPT,), jnp.int32),
            pltpu.VMEM((2, GCH, D), jnp.float32),
            pltpu.SemaphoreType.DMA((2,)),
            pltpu.SemaphoreType.DMA((2,)),
        ],
        compiler_params=pltpu.CompilerParams(needs_layout_passes=False),
    )(_combine_body)
    return k(p0, p1, y)


def _shared_h_body(x_ref, ws1_ref, ws3_ref, h_ref):
    x = x_ref[...].astype(jnp.bfloat16)
    a = jnp.dot(x, ws1_ref[...].astype(jnp.bfloat16),
                preferred_element_type=jnp.float32)
    b = jnp.dot(x, ws3_ref[...].astype(jnp.bfloat16),
                preferred_element_type=jnp.float32)
    h_ref[...] = jax.nn.silu(a) * b


def _shared_out_body(ha_ref, hb_ref, ws2_ref, sig_ref, w0_ref, w1_ref,
                     yg0_ref, yg1_ref, out_ref, *, nfs):
    j = pl.program_id(1)
    h = ha_ref[...]
    part = jnp.dot(h.astype(jnp.bfloat16),
                   ws2_ref[...].astype(jnp.bfloat16),
                   preferred_element_type=jnp.float32)

    @pl.when(j == 0)
    def _():
        out_ref[...] = part

    @pl.when(j != 0)
    def _():
        out_ref[...] += part

    @pl.when(j == nfs - 1)
    def _():
        out_ref[...] = (sig_ref[...] * out_ref[...]
                        + w0_ref[...] * yg0_ref[...]
                        + w1_ref[...] * yg1_ref[...])


FS = 512
NFS = SF // FS
NFA = 8


def _shared_h(x, Ws1, Ws3, start, n):
    Hs = pl.pallas_call(
        _shared_h_body,
        grid=(n,),
        in_specs=[
            pl.BlockSpec((T, D), lambda j: (0, 0)),
            pl.BlockSpec((D, FS), lambda j: (0, j + start)),
            pl.BlockSpec((D, FS), lambda j: (0, j + start)),
        ],
        out_specs=pl.BlockSpec((T, FS), lambda j: (0, j)),
        out_shape=jax.ShapeDtypeStruct((T, n * FS), jnp.float32),
        compiler_params=pltpu.CompilerParams(
            dimension_semantics=("arbitrary",),
            vmem_limit_bytes=112 * 1024 * 1024,
        ),
    )(x, Ws1, Ws3)
    return Hs


def _shared_out(Hsa, Hsb, Ws2, sig, w0, w1, yg0, yg1):
    nfs = NFS
    BT2 = 512
    out = pl.pallas_call(
        functools.partial(_shared_out_body, nfs=nfs),
        grid=(T // BT2, nfs),
        in_specs=[
            pl.BlockSpec((BT2, FS), lambda t, j: (t, j)),
            pl.BlockSpec((BT2, FS), lambda t, j: (t, 0)),
            pl.BlockSpec((FS, D), lambda t, j: (j, 0)),
            pl.BlockSpec((BT2, 1), lambda t, j: (t, 0)),
            pl.BlockSpec((BT2, 1), lambda t, j: (t, 0)),
            pl.BlockSpec((BT2, 1), lambda t, j: (t, 0)),
            pl.BlockSpec((BT2, D), lambda t, j: (t, 0)),
            pl.BlockSpec((BT2, D), lambda t, j: (t, 0)),
        ],
        out_specs=pl.BlockSpec((BT2, D), lambda t, j: (t, 0)),
        out_shape=jax.ShapeDtypeStruct((T, D), jnp.float32),
        compiler_params=pltpu.CompilerParams(
            dimension_semantics=("arbitrary", "arbitrary"),
        ),
    )(Hsa, Hsb, Ws2, sig, w0, w1, yg0, yg1)
    return out


def kernel(x, gate_w, shared_gate_w, Ws1, Ws3, Ws2, W1, W3, W2):
    p0, p1, w0, w1, sig, bexp, valid, nv = _router(x, gate_w, shared_gate_w)
    p0f = p0.reshape(T)
    p1f = p1.reshape(T)
    xg = _dispatch_sc(p0f, p1f, x, nv.reshape(16))
    Hsa = _shared_h(x, Ws1, Ws3, 0, NFS)
    y = _moe_gemm(bexp.reshape(NB), valid.reshape(NB), Hsa, xg, W1, W3, W2)
    yg0, yg1 = _combine_sc(p0f, p1f, y)
    out = _shared_out(Hsa, Hsa, Ws2, sig, w0, w1, yg0, yg1)
    return out


# speedup vs baseline: 1.2480x; 1.0403x over previous
import functools

import jax
import jax.numpy as jnp
from jax import lax
from jax.experimental import pallas as pl
from jax.experimental.pallas import tpu as pltpu
from jax.experimental.pallas import tpu_sc as plsc

D = 2048
F = 1408
E = 16
SF = 5632
T = 2048

B = 256
NB = 32
NSLOTS = NB * B

NEG_INF = -1e30


def _router_body(x_ref, gw_ref, sgw_ref,
                 p0_ref, p1_ref, w0_ref, w1_ref, sig_ref, bexp_ref, valid_ref,
                 nv_ref):
    x = x_ref[...]
    logits = jnp.dot(x, gw_ref[...], preferred_element_type=jnp.float32)
    m = jnp.max(logits, axis=-1, keepdims=True)
    p = jnp.exp(logits - m)
    p = p / jnp.sum(p, axis=-1, keepdims=True)
    lane = lax.broadcasted_iota(jnp.int32, (T, E), 1)
    p1v = jnp.max(p, axis=-1, keepdims=True)
    e0 = jnp.min(jnp.where(p == p1v, lane, E), axis=-1, keepdims=True)
    pm = jnp.where(lane == e0, NEG_INF, p)
    p2v = jnp.max(pm, axis=-1, keepdims=True)
    e1 = jnp.min(jnp.where(pm == p2v, lane, E), axis=-1, keepdims=True)
    s = p1v + p2v
    w0_ref[...] = p1v / s
    w1_ref[...] = p2v / s

    mask = (jnp.where(lane == e0, 1.0, 0.0) + jnp.where(lane == e1, 1.0, 0.0))
    r_t = lax.broadcasted_iota(jnp.int32, (T, T), 0)
    c_t = lax.broadcasted_iota(jnp.int32, (T, T), 1)
    lt = jnp.where(c_t < r_t, 1.0, 0.0)
    ranks = jnp.dot(lt, mask, preferred_element_type=jnp.float32)

    counts = jnp.sum(mask, axis=0, keepdims=True)
    nb = jnp.floor((counts + (B - 1.0)) * (1.0 / B))
    laneE = lax.broadcasted_iota(jnp.int32, (1, E), 1)
    r_e = lax.broadcasted_iota(jnp.int32, (E, E), 0)
    c_e = lax.broadcasted_iota(jnp.int32, (E, E), 1)
    lt16 = jnp.where(r_e < c_e, 1.0, 0.0)
    bstart = jnp.dot(nb, lt16, preferred_element_type=jnp.float32)
    bend = bstart + nb
    off = bstart * float(B)

    sel0 = jnp.where(lane == e0, 1.0, 0.0)
    sel1 = jnp.where(lane == e1, 1.0, 0.0)
    p0 = jnp.sum(sel0 * (off + ranks), axis=-1, keepdims=True)
    p1 = jnp.sum(sel1 * (off + ranks), axis=-1, keepdims=True)
    p0_ref[...] = p0.astype(jnp.int32)
    p1_ref[...] = p1.astype(jnp.int32)

    bi = lax.broadcasted_iota(jnp.int32, (NB, E), 0)
    bend_i = bend.astype(jnp.int32)
    braw = jnp.sum(jnp.where(bi >= bend_i, 1, 0), axis=-1, keepdims=True)
    lastexp = jnp.max(jnp.where(nb > 0, laneE, 0))
    bexp_ref[...] = jnp.minimum(braw, lastexp)
    nbtot = jnp.sum(nb).astype(jnp.int32)
    bvec = lax.broadcasted_iota(jnp.int32, (NB, 1), 0)
    valid_ref[...] = jnp.where(bvec < nbtot, 1, 0).astype(jnp.int32)
    nv_ref[...] = jnp.full((1, 16), B, jnp.int32) * nbtot

    sig_ref[...] = jax.nn.sigmoid(
        jnp.dot(x, sgw_ref[...], preferred_element_type=jnp.float32))


def _router(x, gate_w, shared_gate_w):
    outs = pl.pallas_call(
        _router_body,
        out_shape=(
            jax.ShapeDtypeStruct((T, 1), jnp.int32),
            jax.ShapeDtypeStruct((T, 1), jnp.int32),
            jax.ShapeDtypeStruct((T, 1), jnp.float32),
            jax.ShapeDtypeStruct((T, 1), jnp.float32),
            jax.ShapeDtypeStruct((T, 1), jnp.float32),
            jax.ShapeDtypeStruct((NB, 1), jnp.int32),
            jax.ShapeDtypeStruct((NB, 1), jnp.int32),
            jax.ShapeDtypeStruct((1, 16), jnp.int32),
        ),
    )(x, gate_w, shared_gate_w)
    return outs


def _sc_mesh():
    return plsc.VectorSubcoreMesh(core_axis_name="c", subcore_axis_name="s")


NW = 32
SPT = NSLOTS // NW
GCH = 16
NCH = SPT // GCH


def _dispatch_body(p0_hbm, p1_hbm, x_hbm, nv_hbm, xg_hbm,
                   p0_v, p1_v, table_v, nv_v, rows_v, gsem, osem):
    wid = lax.axis_index("s") * 2 + lax.axis_index("c")
    pltpu.sync_copy(p0_hbm, p0_v)
    pltpu.sync_copy(p1_hbm, p1_v)
    pltpu.sync_copy(nv_hbm, nv_v)
    nvalid = jnp.max(nv_v[...])

    def zero_step(i, _):
        table_v[pl.ds(i * 16, 16)] = jnp.zeros((16,), jnp.int32)
        return 0

    lax.fori_loop(0, NSLOTS // 16, zero_step, 0)

    def scat0(j, _):
        idx = p0_v[pl.ds(j * 16, 16)]
        toks = lax.iota(jnp.int32, 16) + j * 16
        plsc.store_scatter(table_v, [idx], toks)
        return 0

    def scat1(j, _):
        idx = p1_v[pl.ds(j * 16, 16)]
        toks = lax.iota(jnp.int32, 16) + j * 16
        plsc.store_scatter(table_v, [idx], toks)
        return 0

    lax.fori_loop(0, T // 16, scat0, 0)
    lax.fori_loop(0, T // 16, scat1, 0)

    base0 = wid * SPT

    NBUF = 3

    def live(c):
        return (base0 + c * GCH) < nvalid

    def start_gather(c):
        b = c % NBUF
        idx = table_v[pl.ds(base0 + c * GCH, GCH)]
        pltpu.async_copy(x_hbm.at[idx], rows_v.at[b], gsem.at[b])

    for c0 in range(NBUF - 1):

        @pl.when(live(c0))
        def _(c0=c0):
            start_gather(c0)

    for c in range(NCH):
        b = c % NBUF
        g = c + NBUF - 1
        if g < NCH and g - NBUF >= 0:
            @pl.when(live(g - NBUF))
            def _(c=c, g=g):
                pltpu.make_async_copy(
                    rows_v.at[(g - NBUF) % NBUF],
                    xg_hbm.at[pl.ds(base0 + (g - NBUF) * GCH, GCH)],
                    osem.at[(g - NBUF) % NBUF]).wait()
        if g < NCH:

            @pl.when(live(g))
            def _(c=c, g=g):
                start_gather(g)

        @pl.when(live(c))
        def _(c=c):
            pltpu.make_async_copy(x_hbm.at[table_v[pl.ds(0, GCH)]],
                                  rows_v.at[b], gsem.at[b]).wait()
            pltpu.async_copy(rows_v.at[b],
                             xg_hbm.at[pl.ds(base0 + c * GCH, GCH)],
                             osem.at[b])

    for c in range(max(0, NCH - NBUF), NCH):

        @pl.when(live(c))
        def _(c=c):
            pltpu.make_async_copy(rows_v.at[c % NBUF],
                                  xg_hbm.at[pl.ds(base0 + c * GCH, GCH)],
                                  osem.at[c % NBUF]).wait()


def _dispatch_sc(p0, p1, x, nv):
    k = functools.partial(
        pl.kernel,
        out_type=jax.ShapeDtypeStruct((NSLOTS, D), jnp.float32),
        mesh=_sc_mesh(),
        scratch_types=[
            pltpu.VMEM((T,), jnp.int32),
            pltpu.VMEM((T,), jnp.int32),
            pltpu.VMEM((NSLOTS,), jnp.int32),
            pltpu.VMEM((16,), jnp.int32),
            pltpu.VMEM((3, GCH, D), jnp.float32),
            pltpu.SemaphoreType.DMA((3,)),
            pltpu.SemaphoreType.DMA((3,)),
        ],
        compiler_params=pltpu.CompilerParams(needs_layout_passes=False),
    )(_dispatch_body)
    return k(p0, p1, x, nv)


def _moe_h_body(bexp_ref, valid_ref, hdep_ref, xg_ref, w1_ref, w3_ref, h_ref):
    i = pl.program_id(0)

    @pl.when(valid_ref[i] == 1)
    def _():
        xb = xg_ref[...].astype(jnp.bfloat16)
        a1 = jnp.dot(xb, w1_ref[0].astype(jnp.bfloat16),
                     preferred_element_type=jnp.float32)
        a3 = jnp.dot(xb, w3_ref[0].astype(jnp.bfloat16),
                     preferred_element_type=jnp.float32)
        h_ref[...] = jax.nn.silu(a1) * a3


def _moe_y_body(bexp_ref, valid_ref, h_ref, w2_ref, y_ref):
    i = pl.program_id(0)

    @pl.when(valid_ref[i] == 1)
    def _():
        y_ref[...] = jnp.dot(h_ref[...].astype(jnp.bfloat16),
                             w2_ref[0].astype(jnp.bfloat16),
                             preferred_element_type=jnp.float32)


def _moe_gemm(bexp, valid, hdep, xg, W1, W3, W2):
    H = pl.pallas_call(
        _moe_h_body,
        grid_spec=pltpu.PrefetchScalarGridSpec(
            num_scalar_prefetch=2,
            grid=(NB,),
            in_specs=[
                pl.BlockSpec((8, 128), lambda i, be, va: (0, 0)),
                pl.BlockSpec((B, D), lambda i, be, va: (i, 0)),
                pl.BlockSpec((1, D, F), lambda i, be, va: (be[i], 0, 0)),
                pl.BlockSpec((1, D, F), lambda i, be, va: (be[i], 0, 0)),
            ],
            out_specs=pl.BlockSpec((B, F), lambda i, be, va: (i, 0)),
        ),
        out_shape=jax.ShapeDtypeStruct((NSLOTS, F), jnp.float32),
        compiler_params=pltpu.CompilerParams(
            dimension_semantics=("arbitrary",),
            vmem_limit_bytes=63 * 1024 * 1024,
        ),
    )(bexp, valid, hdep, xg, W1, W3)
    y = pl.pallas_call(
        _moe_y_body,
        grid_spec=pltpu.PrefetchScalarGridSpec(
            num_scalar_prefetch=2,
            grid=(NB,),
            in_specs=[
                pl.BlockSpec((B, F), lambda i, be, va: (i, 0)),
                pl.BlockSpec((1, F, D), lambda i, be, va: (be[i], 0, 0)),
            ],
            out_specs=pl.BlockSpec((B, D), lambda i, be, va: (i, 0)),
        ),
        out_shape=jax.ShapeDtypeStruct((NSLOTS, D), jnp.float32),
        compiler_params=pltpu.CompilerParams(
            dimension_semantics=("arbitrary",),
            vmem_limit_bytes=63 * 1024 * 1024,
        ),
    )(bexp, valid, H, W2)
    return y


TPT = T // NW


def _combine_body(p0_hbm, p1_hbm, y_hbm, yg0_hbm, yg1_hbm,
                  p0s, p1s, rows_v, gsem, osem):
    wid = lax.axis_index("s") * 2 + lax.axis_index("c")
    tbase = wid * TPT
    pltpu.sync_copy(p0_hbm.at[pl.ds(tbase, TPT)], p0s)
    pltpu.sync_copy(p1_hbm.at[pl.ds(tbase, TPT)], p1s)

    nq = 2 * (TPT // GCH)

    def chunk(q):
        ps, yg = (p0s, yg0_hbm) if q < nq // 2 else (p1s, yg1_hbm)
        c = q % (nq // 2)
        return ps, yg, c * GCH

    def start_gather(q):
        ps, yg, o = chunk(q)
        b = q % 2
        idx = ps[pl.ds(o, GCH)]
        pltpu.async_copy(y_hbm.at[idx], rows_v.at[b], gsem.at[b])

    start_gather(0)
    for q in range(nq):
        b = q % 2
        if q >= 1:
            psp, ygp, op = chunk(q - 1)
            pltpu.make_async_copy(rows_v.at[1 - b],
                                  ygp.at[pl.ds(tbase + op, GCH)],
                                  osem.at[1 - b]).wait()
        if q + 1 < nq:
            start_gather(q + 1)
        ps, yg, o = chunk(q)
        pltpu.make_async_copy(y_hbm.at[ps[pl.ds(0, GCH)]],
                              rows_v.at[b], gsem.at[b]).wait()
        pltpu.async_copy(rows_v.at[b], yg.at[pl.ds(tbase + o, GCH)],
                         osem.at[b])
    psl, ygl, ol = chunk(nq - 1)
    pltpu.make_async_copy(rows_v.at[(nq - 1) % 2],
                          ygl.at[pl.ds(tbase + ol, GCH)],
                          osem.at[(nq - 1) % 2]).wait()


def _combine_sc(p0, p1, y):
    k = functools.partial(
        pl.kernel,
        out_type=(
            jax.ShapeDtypeStruct((T, D), jnp.float32),
            jax.ShapeDtypeStruct((T, D), jnp.float32),
        ),
        mesh=_sc_mesh(),
        scratch_types=[
            pltpu.VMEM((TPT,), jnp.int32),
            pltpu.VMEM((TPT,), jnp.int32),
            pltpu.VMEM((2, GCH, D), jnp.float32),
            pltpu.SemaphoreType.DMA((2,)),
            pltpu.SemaphoreType.DMA((2,)),
        ],
        compiler_params=pltpu.CompilerParams(needs_layout_passes=False),
    )(_combine_body)
    return k(p0, p1, y)


def _shared_h_body(x_ref, ws1_ref, ws3_ref, h_ref):
    x = x_ref[...].astype(jnp.bfloat16)
    a = jnp.dot(x, ws1_ref[...].astype(jnp.bfloat16),
                preferred_element_type=jnp.float32)
    b = jnp.dot(x, ws3_ref[...].astype(jnp.bfloat16),
                preferred_element_type=jnp.float32)
    h_ref[...] = jax.nn.silu(a) * b


def _shared_acc_body(h_ref, ws2_ref, acc_ref):
    j = pl.program_id(1)
    part = jnp.dot(h_ref[...].astype(jnp.bfloat16),
                   ws2_ref[...].astype(jnp.bfloat16),
                   preferred_element_type=jnp.float32)

    @pl.when(j == 0)
    def _():
        acc_ref[...] = part

    @pl.when(j != 0)
    def _():
        acc_ref[...] += part


def _final_body(acc_ref, sig_ref, w0_ref, w1_ref, yg0_ref, yg1_ref, out_ref):
    out_ref[...] = (sig_ref[...] * acc_ref[...]
                    + w0_ref[...] * yg0_ref[...]
                    + w1_ref[...] * yg1_ref[...])


FS = 512
NFS = SF // FS
NFA = 8


def _shared_h(x, Ws1, Ws3, start, n):
    Hs = pl.pallas_call(
        _shared_h_body,
        grid=(n,),
        in_specs=[
            pl.BlockSpec((T, D), lambda j: (0, 0)),
            pl.BlockSpec((D, FS), lambda j: (0, j + start)),
            pl.BlockSpec((D, FS), lambda j: (0, j + start)),
        ],
        out_specs=pl.BlockSpec((T, FS), lambda j: (0, j)),
        out_shape=jax.ShapeDtypeStruct((T, n * FS), jnp.float32),
        compiler_params=pltpu.CompilerParams(
            dimension_semantics=("arbitrary",),
            vmem_limit_bytes=112 * 1024 * 1024,
        ),
    )(x, Ws1, Ws3)
    return Hs


def _shared_acc(Hs, Ws2):
    BT2 = 1024
    acc = pl.pallas_call(
        _shared_acc_body,
        grid=(T // BT2, NFS),
        in_specs=[
            pl.BlockSpec((BT2, FS), lambda t, j: (t, j)),
            pl.BlockSpec((FS, D), lambda t, j: (j, 0)),
        ],
        out_specs=pl.BlockSpec((BT2, D), lambda t, j: (t, 0)),
        out_shape=jax.ShapeDtypeStruct((T, D), jnp.float32),
        compiler_params=pltpu.CompilerParams(
            dimension_semantics=("arbitrary", "arbitrary"),
        ),
    )(Hs, Ws2)
    return acc


def _final_combine(acc, sig, w0, w1, yg0, yg1):
    BT2 = 512
    out = pl.pallas_call(
        _final_body,
        grid=(T // BT2,),
        in_specs=[
            pl.BlockSpec((BT2, D), lambda t: (t, 0)),
            pl.BlockSpec((BT2, 1), lambda t: (t, 0)),
            pl.BlockSpec((BT2, 1), lambda t: (t, 0)),
            pl.BlockSpec((BT2, 1), lambda t: (t, 0)),
            pl.BlockSpec((BT2, D), lambda t: (t, 0)),
            pl.BlockSpec((BT2, D), lambda t: (t, 0)),
        ],
        out_specs=pl.BlockSpec((BT2, D), lambda t: (t, 0)),
        out_shape=jax.ShapeDtypeStruct((T, D), jnp.float32),
        compiler_params=pltpu.CompilerParams(
            dimension_semantics=("arbitrary",),
        ),
    )(acc, sig, w0, w1, yg0, yg1)
    return out


def kernel(x, gate_w, shared_gate_w, Ws1, Ws3, Ws2, W1, W3, W2):
    p0, p1, w0, w1, sig, bexp, valid, nv = _router(x, gate_w, shared_gate_w)
    p0f = p0.reshape(T)
    p1f = p1.reshape(T)
    xg = _dispatch_sc(p0f, p1f, x, nv.reshape(16))
    Hs = _shared_h(x, Ws1, Ws3, 0, NFS)
    y = _moe_gemm(bexp.reshape(NB), valid.reshape(NB), Hs, xg, W1, W3, W2)
    acc = _shared_acc(Hs, Ws2)
    yg0, yg1 = _combine_sc(p0f, p1f, y)
    out = _final_combine(acc, sig, w0, w1, yg0, yg1)
    return out


# speedup vs baseline: 1.2941x; 1.0369x over previous
import functools

import jax
import jax.numpy as jnp
from jax import lax
from jax.experimental import pallas as pl
from jax.experimental.pallas import tpu as pltpu
from jax.experimental.pallas import tpu_sc as plsc

D = 2048
F = 1408
E = 16
SF = 5632
T = 2048

B = 256
NB = 32
NSLOTS = NB * B

NEG_INF = -1e30


def _router_body(x_ref, gw_ref, sgw_ref,
                 p0_ref, p1_ref, w0_ref, w1_ref, sig_ref, bexp_ref, valid_ref,
                 nv_ref):
    x = x_ref[...]
    logits = jnp.dot(x, gw_ref[...], preferred_element_type=jnp.float32)
    m = jnp.max(logits, axis=-1, keepdims=True)
    p = jnp.exp(logits - m)
    p = p / jnp.sum(p, axis=-1, keepdims=True)
    lane = lax.broadcasted_iota(jnp.int32, (T, E), 1)
    p1v = jnp.max(p, axis=-1, keepdims=True)
    e0 = jnp.min(jnp.where(p == p1v, lane, E), axis=-1, keepdims=True)
    pm = jnp.where(lane == e0, NEG_INF, p)
    p2v = jnp.max(pm, axis=-1, keepdims=True)
    e1 = jnp.min(jnp.where(pm == p2v, lane, E), axis=-1, keepdims=True)
    s = p1v + p2v
    w0_ref[...] = p1v / s
    w1_ref[...] = p2v / s

    mask = (jnp.where(lane == e0, 1.0, 0.0) + jnp.where(lane == e1, 1.0, 0.0))
    r_t = lax.broadcasted_iota(jnp.int32, (T, T), 0)
    c_t = lax.broadcasted_iota(jnp.int32, (T, T), 1)
    lt = jnp.where(c_t < r_t, 1.0, 0.0)
    ranks = jnp.dot(lt, mask, preferred_element_type=jnp.float32)

    counts = jnp.sum(mask, axis=0, keepdims=True)
    nb = jnp.floor((counts + (B - 1.0)) * (1.0 / B))
    laneE = lax.broadcasted_iota(jnp.int32, (1, E), 1)
    r_e = lax.broadcasted_iota(jnp.int32, (E, E), 0)
    c_e = lax.broadcasted_iota(jnp.int32, (E, E), 1)
    lt16 = jnp.where(r_e < c_e, 1.0, 0.0)
    bstart = jnp.dot(nb, lt16, preferred_element_type=jnp.float32)
    bend = bstart + nb
    off = bstart * float(B)

    sel0 = jnp.where(lane == e0, 1.0, 0.0)
    sel1 = jnp.where(lane == e1, 1.0, 0.0)
    p0 = jnp.sum(sel0 * (off + ranks), axis=-1, keepdims=True)
    p1 = jnp.sum(sel1 * (off + ranks), axis=-1, keepdims=True)
    p0_ref[...] = p0.astype(jnp.int32)
    p1_ref[...] = p1.astype(jnp.int32)

    bi = lax.broadcasted_iota(jnp.int32, (NB, E), 0)
    bend_i = bend.astype(jnp.int32)
    braw = jnp.sum(jnp.where(bi >= bend_i, 1, 0), axis=-1, keepdims=True)
    lastexp = jnp.max(jnp.where(nb > 0, laneE, 0))
    bexp_ref[...] = jnp.minimum(braw, lastexp)
    nbtot = jnp.sum(nb).astype(jnp.int32)
    bvec = lax.broadcasted_iota(jnp.int32, (NB, 1), 0)
    valid_ref[...] = jnp.where(bvec < nbtot, 1, 0).astype(jnp.int32)
    nv_ref[...] = jnp.full((1, 16), B, jnp.int32) * nbtot

    sig_ref[...] = jax.nn.sigmoid(
        jnp.dot(x, sgw_ref[...], preferred_element_type=jnp.float32))


def _router(x, gate_w, shared_gate_w):
    outs = pl.pallas_call(
        _router_body,
        out_shape=(
            jax.ShapeDtypeStruct((T, 1), jnp.int32),
            jax.ShapeDtypeStruct((T, 1), jnp.int32),
            jax.ShapeDtypeStruct((T, 1), jnp.float32),
            jax.ShapeDtypeStruct((T, 1), jnp.float32),
            jax.ShapeDtypeStruct((T, 1), jnp.float32),
            jax.ShapeDtypeStruct((NB, 1), jnp.int32),
            jax.ShapeDtypeStruct((NB, 1), jnp.int32),
            jax.ShapeDtypeStruct((1, 16), jnp.int32),
        ),
    )(x, gate_w, shared_gate_w)
    return outs


def _sc_mesh():
    return plsc.VectorSubcoreMesh(core_axis_name="c", subcore_axis_name="s")


NW = 32
SPT = NSLOTS // NW
GCH = 16
NCH = SPT // GCH


def _dispatch_body(p0_hbm, p1_hbm, x_hbm, nv_hbm, xg_hbm,
                   p0_v, p1_v, table_v, nv_v, rows_v, gsem, osem):
    wid = lax.axis_index("s") * 2 + lax.axis_index("c")
    pltpu.sync_copy(p0_hbm, p0_v)
    pltpu.sync_copy(p1_hbm, p1_v)
    pltpu.sync_copy(nv_hbm, nv_v)
    nvalid = jnp.max(nv_v[...])

    def zero_step(i, _):
        table_v[pl.ds(i * 16, 16)] = jnp.zeros((16,), jnp.int32)
        return 0

    lax.fori_loop(0, NSLOTS // 16, zero_step, 0)

    def scat0(j, _):
        idx = p0_v[pl.ds(j * 16, 16)]
        toks = lax.iota(jnp.int32, 16) + j * 16
        plsc.store_scatter(table_v, [idx], toks)
        return 0

    def scat1(j, _):
        idx = p1_v[pl.ds(j * 16, 16)]
        toks = lax.iota(jnp.int32, 16) + j * 16
        plsc.store_scatter(table_v, [idx], toks)
        return 0

    lax.fori_loop(0, T // 16, scat0, 0)
    lax.fori_loop(0, T // 16, scat1, 0)

    base0 = wid * SPT

    NBUF = 3

    def live(c):
        return (base0 + c * GCH) < nvalid

    def start_gather(c):
        b = c % NBUF
        idx = table_v[pl.ds(base0 + c * GCH, GCH)]
        pltpu.async_copy(x_hbm.at[idx], rows_v.at[b], gsem.at[b])

    for c0 in range(NBUF - 1):

        @pl.when(live(c0))
        def _(c0=c0):
            start_gather(c0)

    for c in range(NCH):
        b = c % NBUF
        g = c + NBUF - 1
        if g < NCH and g - NBUF >= 0:
            @pl.when(live(g - NBUF))
            def _(c=c, g=g):
                pltpu.make_async_copy(
                    rows_v.at[(g - NBUF) % NBUF],
                    xg_hbm.at[pl.ds(base0 + (g - NBUF) * GCH, GCH)],
                    osem.at[(g - NBUF) % NBUF]).wait()
        if g < NCH:

            @pl.when(live(g))
            def _(c=c, g=g):
                start_gather(g)

        @pl.when(live(c))
        def _(c=c):
            pltpu.make_async_copy(x_hbm.at[table_v[pl.ds(0, GCH)]],
                                  rows_v.at[b], gsem.at[b]).wait()
            pltpu.async_copy(rows_v.at[b],
                             xg_hbm.at[pl.ds(base0 + c * GCH, GCH)],
                             osem.at[b])

    for c in range(max(0, NCH - NBUF), NCH):

        @pl.when(live(c))
        def _(c=c):
            pltpu.make_async_copy(rows_v.at[c % NBUF],
                                  xg_hbm.at[pl.ds(base0 + c * GCH, GCH)],
                                  osem.at[c % NBUF]).wait()


def _dispatch_sc(p0, p1, x, nv):
    k = functools.partial(
        pl.kernel,
        out_type=jax.ShapeDtypeStruct((NSLOTS, D), jnp.float32),
        mesh=_sc_mesh(),
        scratch_types=[
            pltpu.VMEM((T,), jnp.int32),
            pltpu.VMEM((T,), jnp.int32),
            pltpu.VMEM((NSLOTS,), jnp.int32),
            pltpu.VMEM((16,), jnp.int32),
            pltpu.VMEM((3, GCH, D), jnp.float32),
            pltpu.SemaphoreType.DMA((3,)),
            pltpu.SemaphoreType.DMA((3,)),
        ],
        compiler_params=pltpu.CompilerParams(needs_layout_passes=False),
    )(_dispatch_body)
    return k(p0, p1, x, nv)


def _moe_h_body(bexp_ref, valid_ref, hdep_ref, xg_ref, w1_ref, w3_ref, h_ref):
    i = pl.program_id(0)

    @pl.when(valid_ref[i] == 1)
    def _():
        xb = xg_ref[...].astype(jnp.bfloat16)
        a1 = jnp.dot(xb, w1_ref[0].astype(jnp.bfloat16),
                     preferred_element_type=jnp.float32)
        a3 = jnp.dot(xb, w3_ref[0].astype(jnp.bfloat16),
                     preferred_element_type=jnp.float32)
        h_ref[...] = (jax.nn.silu(a1) * a3).astype(jnp.bfloat16)


def _moe_y_body(bexp_ref, valid_ref, h_ref, w2_ref, y_ref):
    i = pl.program_id(0)

    @pl.when(valid_ref[i] == 1)
    def _():
        y_ref[...] = jnp.dot(h_ref[...],
                             w2_ref[0].astype(jnp.bfloat16),
                             preferred_element_type=jnp.float32)


def _moe_gemm(bexp, valid, hdep, xg, W1, W3, W2):
    H = pl.pallas_call(
        _moe_h_body,
        grid_spec=pltpu.PrefetchScalarGridSpec(
            num_scalar_prefetch=2,
            grid=(NB,),
            in_specs=[
                pl.BlockSpec((16, 128), lambda i, be, va: (0, 0)),
                pl.BlockSpec((B, D), lambda i, be, va: (i, 0)),
                pl.BlockSpec((1, D, F), lambda i, be, va: (be[i], 0, 0)),
                pl.BlockSpec((1, D, F), lambda i, be, va: (be[i], 0, 0)),
            ],
            out_specs=pl.BlockSpec((B, F), lambda i, be, va: (i, 0)),
        ),
        out_shape=jax.ShapeDtypeStruct((NSLOTS, F), jnp.bfloat16),
        compiler_params=pltpu.CompilerParams(
            dimension_semantics=("arbitrary",),
            vmem_limit_bytes=63 * 1024 * 1024,
        ),
    )(bexp, valid, hdep, xg, W1, W3)
    y = pl.pallas_call(
        _moe_y_body,
        grid_spec=pltpu.PrefetchScalarGridSpec(
            num_scalar_prefetch=2,
            grid=(NB,),
            in_specs=[
                pl.BlockSpec((B, F), lambda i, be, va: (i, 0)),
                pl.BlockSpec((1, F, D), lambda i, be, va: (be[i], 0, 0)),
            ],
            out_specs=pl.BlockSpec((B, D), lambda i, be, va: (i, 0)),
        ),
        out_shape=jax.ShapeDtypeStruct((NSLOTS, D), jnp.float32),
        compiler_params=pltpu.CompilerParams(
            dimension_semantics=("arbitrary",),
            vmem_limit_bytes=63 * 1024 * 1024,
        ),
    )(bexp, valid, H, W2)
    return y


TPT = T // NW


def _combine_body(p0_hbm, p1_hbm, y_hbm, yg0_hbm, yg1_hbm,
                  p0s, p1s, rows_v, gsem, osem):
    wid = lax.axis_index("s") * 2 + lax.axis_index("c")
    tbase = wid * TPT
    pltpu.sync_copy(p0_hbm.at[pl.ds(tbase, TPT)], p0s)
    pltpu.sync_copy(p1_hbm.at[pl.ds(tbase, TPT)], p1s)

    nq = 2 * (TPT // GCH)

    def chunk(q):
        ps, yg = (p0s, yg0_hbm) if q < nq // 2 else (p1s, yg1_hbm)
        c = q % (nq // 2)
        return ps, yg, c * GCH

    def start_gather(q):
        ps, yg, o = chunk(q)
        b = q % 2
        idx = ps[pl.ds(o, GCH)]
        pltpu.async_copy(y_hbm.at[idx], rows_v.at[b], gsem.at[b])

    start_gather(0)
    for q in range(nq):
        b = q % 2
        if q >= 1:
            psp, ygp, op = chunk(q - 1)
            pltpu.make_async_copy(rows_v.at[1 - b],
                                  ygp.at[pl.ds(tbase + op, GCH)],
                                  osem.at[1 - b]).wait()
        if q + 1 < nq:
            start_gather(q + 1)
        ps, yg, o = chunk(q)
        pltpu.make_async_copy(y_hbm.at[ps[pl.ds(0, GCH)]],
                              rows_v.at[b], gsem.at[b]).wait()
        pltpu.async_copy(rows_v.at[b], yg.at[pl.ds(tbase + o, GCH)],
                         osem.at[b])
    psl, ygl, ol = chunk(nq - 1)
    pltpu.make_async_copy(rows_v.at[(nq - 1) % 2],
                          ygl.at[pl.ds(tbase + ol, GCH)],
                          osem.at[(nq - 1) % 2]).wait()


def _combine_sc(p0, p1, y):
    k = functools.partial(
        pl.kernel,
        out_type=(
            jax.ShapeDtypeStruct((T, D), jnp.float32),
            jax.ShapeDtypeStruct((T, D), jnp.float32),
        ),
        mesh=_sc_mesh(),
        scratch_types=[
            pltpu.VMEM((TPT,), jnp.int32),
            pltpu.VMEM((TPT,), jnp.int32),
            pltpu.VMEM((2, GCH, D), jnp.float32),
            pltpu.SemaphoreType.DMA((2,)),
            pltpu.SemaphoreType.DMA((2,)),
        ],
        compiler_params=pltpu.CompilerParams(needs_layout_passes=False),
    )(_combine_body)
    return k(p0, p1, y)


def _shared_h_body(x_ref, ws1_ref, ws3_ref, h_ref):
    x = x_ref[...].astype(jnp.bfloat16)
    a = jnp.dot(x, ws1_ref[...].astype(jnp.bfloat16),
                preferred_element_type=jnp.float32)
    b = jnp.dot(x, ws3_ref[...].astype(jnp.bfloat16),
                preferred_element_type=jnp.float32)
    h_ref[...] = (jax.nn.silu(a) * b).astype(jnp.bfloat16)


def _shared_acc_body(h_ref, ws2_ref, acc_ref):
    j = pl.program_id(1)
    part = jnp.dot(h_ref[...],
                   ws2_ref[...].astype(jnp.bfloat16),
                   preferred_element_type=jnp.float32)

    @pl.when(j == 0)
    def _():
        acc_ref[...] = part

    @pl.when(j != 0)
    def _():
        acc_ref[...] += part


def _final_body(acc_ref, sig_ref, w0_ref, w1_ref, yg0_ref, yg1_ref, out_ref):
    out_ref[...] = (sig_ref[...] * acc_ref[...]
                    + w0_ref[...] * yg0_ref[...]
                    + w1_ref[...] * yg1_ref[...])


FS = 512
NFS = SF // FS
NFA = 8


def _shared_h(x, Ws1, Ws3, start, n):
    Hs = pl.pallas_call(
        _shared_h_body,
        grid=(n,),
        in_specs=[
            pl.BlockSpec((T, D), lambda j: (0, 0)),
            pl.BlockSpec((D, FS), lambda j: (0, j + start)),
            pl.BlockSpec((D, FS), lambda j: (0, j + start)),
        ],
        out_specs=pl.BlockSpec((T, FS), lambda j: (0, j)),
        out_shape=jax.ShapeDtypeStruct((T, n * FS), jnp.bfloat16),
        compiler_params=pltpu.CompilerParams(
            dimension_semantics=("arbitrary",),
            vmem_limit_bytes=112 * 1024 * 1024,
        ),
    )(x, Ws1, Ws3)
    return Hs


def _shared_acc(Hs, Ws2):
    BT2 = 1024
    acc = pl.pallas_call(
        _shared_acc_body,
        grid=(T // BT2, NFS),
        in_specs=[
            pl.BlockSpec((BT2, FS), lambda t, j: (t, j)),
            pl.BlockSpec((FS, D), lambda t, j: (j, 0)),
        ],
        out_specs=pl.BlockSpec((BT2, D), lambda t, j: (t, 0)),
        out_shape=jax.ShapeDtypeStruct((T, D), jnp.float32),
        compiler_params=pltpu.CompilerParams(
            dimension_semantics=("arbitrary", "arbitrary"),
        ),
    )(Hs, Ws2)
    return acc


def _final_combine(acc, sig, w0, w1, yg0, yg1):
    BT2 = 512
    out = pl.pallas_call(
        _final_body,
        grid=(T // BT2,),
        in_specs=[
            pl.BlockSpec((BT2, D), lambda t: (t, 0)),
            pl.BlockSpec((BT2, 1), lambda t: (t, 0)),
            pl.BlockSpec((BT2, 1), lambda t: (t, 0)),
            pl.BlockSpec((BT2, 1), lambda t: (t, 0)),
            pl.BlockSpec((BT2, D), lambda t: (t, 0)),
            pl.BlockSpec((BT2, D), lambda t: (t, 0)),
        ],
        out_specs=pl.BlockSpec((BT2, D), lambda t: (t, 0)),
        out_shape=jax.ShapeDtypeStruct((T, D), jnp.float32),
        compiler_params=pltpu.CompilerParams(
            dimension_semantics=("arbitrary",),
        ),
    )(acc, sig, w0, w1, yg0, yg1)
    return out


def kernel(x, gate_w, shared_gate_w, Ws1, Ws3, Ws2, W1, W3, W2):
    p0, p1, w0, w1, sig, bexp, valid, nv = _router(x, gate_w, shared_gate_w)
    p0f = p0.reshape(T)
    p1f = p1.reshape(T)
    xg = _dispatch_sc(p0f, p1f, x, nv.reshape(16))
    Hs = _shared_h(x, Ws1, Ws3, 0, NFS)
    y = _moe_gemm(bexp.reshape(NB), valid.reshape(NB), Hs, xg, W1, W3, W2)
    acc = _shared_acc(Hs, Ws2)
    yg0, yg1 = _combine_sc(p0f, p1f, y)
    out = _final_combine(acc, sig, w0, w1, yg0, yg1)
    return out
